# baseline (device time: 28787 ns/iter reference)
import jax
import jax.numpy as jnp
from jax import lax
from jax.experimental import pallas as pl
from jax.experimental.pallas import tpu as pltpu


def kernel(x, router, W1, W2):
    T_loc, D = x.shape
    E_loc, _, F = W1.shape
    E = 2 * E_loc
    T = 2 * T_loc

    def body(x_ref, r_ref, w1_ref, w2_ref, out_ref,
             xs_ref, xr_ref, rr_ref, ws_ref, wr_ref,
             xfull_ref, wfull_ref, acc_ref, cs_ref, cr_ref,
             send_sems, recv_sems):
        my_x = lax.axis_index("x")
        my_y = lax.axis_index("y")
        my_z = lax.axis_index("z")
        peer = (my_x, 1 - my_y, my_z)
        is0 = my_y == 0

        barrier = pltpu.get_barrier_semaphore()
        pl.semaphore_signal(barrier, inc=1, device_id=peer,
                            device_id_type=pl.DeviceIdType.MESH)
        pl.semaphore_wait(barrier, 1)

        xs_ref[...] = x_ref[...].astype(jnp.bfloat16)
        rdma_x = pltpu.make_async_remote_copy(
            src_ref=xs_ref, dst_ref=xr_ref,
            send_sem=send_sems.at[0], recv_sem=recv_sems.at[0],
            device_id=peer, device_id_type=pl.DeviceIdType.MESH)
        rdma_x.start()
        rdma_r = pltpu.make_async_remote_copy(
            src_ref=r_ref, dst_ref=rr_ref,
            send_sem=send_sems.at[1], recv_sem=recv_sems.at[1],
            device_id=peer, device_id_type=pl.DeviceIdType.MESH)
        rdma_r.start()
        rdma_r.wait()

        r_mine = r_ref[...]
        r_peer = rr_ref[...]
        r_lo = jnp.where(is0, r_mine, r_peer)
        r_hi = jnp.where(is0, r_peer, r_mine)
        xf = x_ref[...]
        g_lo = jnp.dot(xf, r_lo, precision=lax.Precision.HIGHEST,
                       preferred_element_type=jnp.float32)
        g_hi = jnp.dot(xf, r_hi, precision=lax.Precision.HIGHEST,
                       preferred_element_type=jnp.float32)
        gates = jnp.concatenate([g_lo, g_hi], axis=1)

        cols = [gates[:, e:e + 1] for e in range(E)]
        cnt = []
        for e in range(E):
            c = jnp.zeros((T_loc, 1), jnp.float32)
            for j in range(E):
                if j == e:
                    continue
                beats = (cols[j] > cols[e]) | (
                    (cols[j] == cols[e]) & (j < e))
                c = c + beats.astype(jnp.float32)
            cnt.append(c)
        m1 = sum(jnp.where(cnt[e] == 0, cols[e], 0.0) for e in range(E))
        m2 = sum(jnp.where(cnt[e] == 1, cols[e], 0.0) for e in range(E))
        denom = 1.0 + jnp.exp(m2 - m1)
        wcols = [
            jnp.where(cnt[e] < 2, jnp.exp(cols[e] - m1) / denom, 0.0)
            for e in range(E)
        ]
        ws_ref[...] = jnp.concatenate(wcols, axis=1)

        rdma_w = pltpu.make_async_remote_copy(
            src_ref=ws_ref, dst_ref=wr_ref,
            send_sem=send_sems.at[2], recv_sem=recv_sems.at[2],
            device_id=peer, device_id_type=pl.DeviceIdType.MESH)
        rdma_w.start()
        rdma_x.wait()

        xfull_ref[pl.ds(my_y * T_loc, T_loc), :] = xs_ref[...]
        xfull_ref[pl.ds((1 - my_y) * T_loc, T_loc), :] = xr_ref[...]
        rdma_w.wait()

        wm = ws_ref[...]
        wp = wr_ref[...]
        for e_loc in range(E_loc):
            lo = wm[:, e_loc:e_loc + 1]
            hi = wm[:, E_loc + e_loc:E_loc + e_loc + 1]
            wm_e = jnp.where(is0, lo, hi)
            lo_p = wp[:, e_loc:e_loc + 1]
            hi_p = wp[:, E_loc + e_loc:E_loc + e_loc + 1]
            wp_e = jnp.where(is0, lo_p, hi_p)
            wfull_ref[pl.ds(my_y * T_loc, T_loc), e_loc:e_loc + 1] = wm_e
            wfull_ref[pl.ds((1 - my_y) * T_loc, T_loc),
                      e_loc:e_loc + 1] = wp_e

        xfull = xfull_ref[...]
        acc = jnp.zeros((T, D), jnp.float32)
        for e_loc in range(E_loc):
            h = jnp.maximum(
                jnp.dot(xfull, w1_ref[e_loc].astype(jnp.bfloat16),
                        preferred_element_type=jnp.float32), 0.0)
            y_e = jnp.dot(h.astype(jnp.bfloat16),
                          w2_ref[e_loc].astype(jnp.bfloat16),
                          preferred_element_type=jnp.float32)
            acc = acc + y_e * wfull_ref[:, e_loc:e_loc + 1]
        acc_ref[...] = acc

        cs_ref[...] = acc_ref[pl.ds((1 - my_y) * T_loc, T_loc),
                              :].astype(jnp.bfloat16)
        rdma_c = pltpu.make_async_remote_copy(
            src_ref=cs_ref, dst_ref=cr_ref,
            send_sem=send_sems.at[3], recv_sem=recv_sems.at[3],
            device_id=peer, device_id_type=pl.DeviceIdType.MESH)
        rdma_c.start()
        rdma_c.wait()
        out_ref[...] = (acc_ref[pl.ds(my_y * T_loc, T_loc), :]
                        + cr_ref[...].astype(jnp.float32))

    return pl.pallas_call(
        body,
        out_shape=jax.ShapeDtypeStruct((T_loc, D), jnp.float32),
        in_specs=[pl.BlockSpec(memory_space=pltpu.VMEM)] * 4,
        out_specs=pl.BlockSpec(memory_space=pltpu.VMEM),
        scratch_shapes=[
            pltpu.VMEM((T_loc, D), jnp.bfloat16),
            pltpu.VMEM((T_loc, D), jnp.bfloat16),
            pltpu.VMEM((D, E_loc), jnp.float32),
            pltpu.VMEM((T_loc, E), jnp.float32),
            pltpu.VMEM((T_loc, E), jnp.float32),
            pltpu.VMEM((T, D), jnp.bfloat16),
            pltpu.VMEM((T, E_loc), jnp.float32),
            pltpu.VMEM((T, D), jnp.float32),
            pltpu.VMEM((T_loc, D), jnp.bfloat16),
            pltpu.VMEM((T_loc, D), jnp.bfloat16),
            pltpu.SemaphoreType.DMA((4,)),
            pltpu.SemaphoreType.DMA((4,)),
        ],
        compiler_params=pltpu.CompilerParams(collective_id=0),
    )(x, router, W1, W2)


# device time: 25585 ns/iter; 1.1252x vs baseline; 1.1252x over previous
import jax
import jax.numpy as jnp
from jax import lax
from jax.experimental import pallas as pl
from jax.experimental.pallas import tpu as pltpu


def kernel(x, router, W1, W2):
    T_loc, D = x.shape
    E_loc, _, F = W1.shape
    E = 2 * E_loc

    def body(x_ref, r_ref, w1_hbm, w2_hbm, out_ref,
             xs_ref, xr_ref, rr_ref, ws_ref, wr_ref,
             w1f_ref, w2f_ref, w1b_ref, w2b_ref,
             cs_ref, cr_ref,
             local_sems, send_sems, recv_sems):
        my_x = lax.axis_index("x")
        my_y = lax.axis_index("y")
        my_z = lax.axis_index("z")
        peer = (my_x, 1 - my_y, my_z)
        is0 = my_y == 0

        dma_w1 = pltpu.make_async_copy(w1_hbm, w1f_ref, local_sems.at[0])
        dma_w2 = pltpu.make_async_copy(w2_hbm, w2f_ref, local_sems.at[1])
        dma_w1.start()
        dma_w2.start()

        barrier = pltpu.get_barrier_semaphore()
        pl.semaphore_signal(barrier, inc=1, device_id=peer,
                            device_id_type=pl.DeviceIdType.MESH)
        pl.semaphore_wait(barrier, 1)

        rdma_r = pltpu.make_async_remote_copy(
            src_ref=r_ref, dst_ref=rr_ref,
            send_sem=send_sems.at[1], recv_sem=recv_sems.at[1],
            device_id=peer, device_id_type=pl.DeviceIdType.MESH)
        rdma_r.start()
        xs_ref[...] = x_ref[...].astype(jnp.bfloat16)
        rdma_x = pltpu.make_async_remote_copy(
            src_ref=xs_ref, dst_ref=xr_ref,
            send_sem=send_sems.at[0], recv_sem=recv_sems.at[0],
            device_id=peer, device_id_type=pl.DeviceIdType.MESH)
        rdma_x.start()

        dma_w1.wait()
        w1b_ref[...] = w1f_ref[...].astype(jnp.bfloat16)
        dma_w2.wait()
        w2b_ref[...] = w2f_ref[...].astype(jnp.bfloat16)

        rdma_r.wait()
        r_mine = r_ref[...]
        r_peer = rr_ref[...]
        r_all = jnp.concatenate(
            [jnp.where(is0, r_mine, r_peer), jnp.where(is0, r_peer, r_mine)],
            axis=1)
        gates = jnp.dot(x_ref[...], r_all, precision=lax.Precision.HIGHEST,
                        preferred_element_type=jnp.float32)

        cols = [gates[:, e:e + 1] for e in range(E)]
        cnt = []
        for e in range(E):
            c = jnp.zeros((T_loc, 1), jnp.float32)
            for j in range(E):
                if j == e:
                    continue
                beats = cols[j] > cols[e] if j > e else cols[j] >= cols[e]
                c = c + beats.astype(jnp.float32)
            cnt.append(c)
        m1 = sum(jnp.where(cnt[e] == 0, cols[e], 0.0) for e in range(E))
        m2 = sum(jnp.where(cnt[e] == 1, cols[e], 0.0) for e in range(E))
        denom = 1.0 + jnp.exp(m2 - m1)
        wcols = [
            jnp.where(cnt[e] < 2, jnp.exp(cols[e] - m1) / denom, 0.0)
            for e in range(E)
        ]
        ws_ref[...] = jnp.concatenate(wcols, axis=1)

        rdma_w = pltpu.make_async_remote_copy(
            src_ref=ws_ref, dst_ref=wr_ref,
            send_sem=send_sems.at[2], recv_sem=recv_sems.at[2],
            device_id=peer, device_id_type=pl.DeviceIdType.MESH)
        rdma_w.start()

        def expert_half(tokens_bf16, weights):
            acc = jnp.zeros((T_loc, D), jnp.float32)
            for e_loc in range(E_loc):
                lo = weights[:, e_loc:e_loc + 1]
                hi = weights[:, E_loc + e_loc:E_loc + e_loc + 1]
                w_e = jnp.where(is0, lo, hi)
                h = jnp.maximum(
                    jnp.dot(tokens_bf16, w1b_ref[e_loc],
                            preferred_element_type=jnp.float32), 0.0)
                y_e = jnp.dot(h.astype(jnp.bfloat16), w2b_ref[e_loc],
                              preferred_element_type=jnp.float32)
                acc = acc + y_e * w_e
            return acc

        rdma_x.wait()
        rdma_w.wait()
        cs_ref[...] = expert_half(xr_ref[...], wr_ref[...]).astype(
            jnp.bfloat16)
        rdma_c = pltpu.make_async_remote_copy(
            src_ref=cs_ref, dst_ref=cr_ref,
            send_sem=send_sems.at[3], recv_sem=recv_sems.at[3],
            device_id=peer, device_id_type=pl.DeviceIdType.MESH)
        rdma_c.start()

        acc_mine = expert_half(xs_ref[...], ws_ref[...])
        rdma_c.wait()
        out_ref[...] = acc_mine + cr_ref[...].astype(jnp.float32)

    return pl.pallas_call(
        body,
        out_shape=jax.ShapeDtypeStruct((T_loc, D), jnp.float32),
        in_specs=[
            pl.BlockSpec(memory_space=pltpu.VMEM),
            pl.BlockSpec(memory_space=pltpu.VMEM),
            pl.BlockSpec(memory_space=pl.ANY),
            pl.BlockSpec(memory_space=pl.ANY),
        ],
        out_specs=pl.BlockSpec(memory_space=pltpu.VMEM),
        scratch_shapes=[
            pltpu.VMEM((T_loc, D), jnp.bfloat16),
            pltpu.VMEM((T_loc, D), jnp.bfloat16),
            pltpu.VMEM((D, E_loc), jnp.float32),
            pltpu.VMEM((T_loc, E), jnp.float32),
            pltpu.VMEM((T_loc, E), jnp.float32),
            pltpu.VMEM((E_loc, D, F), jnp.float32),
            pltpu.VMEM((E_loc, F, D), jnp.float32),
            pltpu.VMEM((E_loc, D, F), jnp.bfloat16),
            pltpu.VMEM((E_loc, F, D), jnp.bfloat16),
            pltpu.VMEM((T_loc, D), jnp.bfloat16),
            pltpu.VMEM((T_loc, D), jnp.bfloat16),
            pltpu.SemaphoreType.DMA((2,)),
            pltpu.SemaphoreType.DMA((4,)),
            pltpu.SemaphoreType.DMA((4,)),
        ],
        compiler_params=pltpu.CompilerParams(collective_id=0),
    )(x, router, W1, W2)
